# Initial kernel scaffold; baseline (speedup 1.0000x reference)
import functools

import jax
import jax.numpy as jnp
from jax import lax
from jax.experimental import pallas as pl
from jax.experimental.pallas import tpu as pltpu
from jax.experimental.pallas import tpu_sc as plsc

L = 16
NC = 2
NS = 16
NW = NC * NS
C = 128


def _tc_prep(ns_ref, w_ref, at_ref, h_ref, st_ref):
    h = jnp.dot(ns_ref[...], w_ref[...], preferred_element_type=jnp.float32)
    h_ref[...] = h
    st_ref[...] = lax.dot_general(
        at_ref[...], h, (((1,), (1,)), ((), ())),
        preferred_element_type=jnp.float32)


def _tc_finish(u_ref, as_ref, o_ref):
    u = u_ref[0] + u_ref[1]
    d = (as_ref[0] + as_ref[1])[:, None]
    o_ref[...] = jnp.where(d > 0.0, u / jnp.where(d > 0.0, d, 1.0), 0.0)


def _sc_body(h_hbm, st_hbm, dst_hbm, src_hbm, u_out, a_out,
             s_loc, t_loc, dst_loc, src_loc, att_loc, hrows, wrows,
             zbuf, zatt, u_sh, as_sh, sem, *, kj, np_):
    cid = lax.axis_index("c")
    sid = lax.axis_index("s")
    w = cid * NS + sid
    rows = np_ // NS
    nz = rows // L

    z16 = jnp.zeros((L,), jnp.float32)

    def zb(i, c_):
        for k in range(8):
            zbuf[i, pl.ds(k * L, L)] = z16
        return c_
    lax.fori_loop(0, L, zb, 0)

    def za(i, c_):
        zatt[pl.ds(i * L, L)] = z16
        return c_
    lax.fori_loop(0, rows // L, za, 0)

    def zu(i, c_):
        pltpu.sync_copy(zbuf, u_sh.at[pl.ds(sid * rows + i * L, L)])
        return c_
    lax.fori_loop(0, nz, zu, 0)
    pltpu.sync_copy(zatt, as_sh.at[pl.ds(sid * rows, rows)])

    pltpu.sync_copy(st_hbm.at[0], s_loc)
    pltpu.sync_copy(st_hbm.at[1], t_loc)
    pltpu.sync_copy(dst_hbm.at[pl.ds(w * kj, kj)], dst_loc)
    pltpu.sync_copy(src_hbm.at[pl.ds(w * kj, kj)], src_loc)

    plsc.subcore_barrier()

    def body_a(j, c_):
        for k in range(C // L):
            dv = dst_loc[j, pl.ds(k * L, L)]
            sv = src_loc[j, pl.ds(k * L, L)]
            x = plsc.load_gather(s_loc, [dv]) + plsc.load_gather(t_loc, [sv])
            x = jnp.maximum(x, 0.2 * x)
            x = jnp.minimum(jnp.maximum(x, -2.0), 2.0)
            att_loc[j, pl.ds(k * L, L)] = jnp.exp(x)
        pltpu.sync_copy(att_loc.at[j], as_sh.at[dst_loc.at[j]], add=True)
        return c_
    lax.fori_loop(0, kj, body_a, 0)

    def body_b(j, c_):
        pltpu.async_copy(h_hbm.at[src_loc.at[j]], hrows, sem).wait()

        def wbody(i, c2):
            a = att_loc[j, i]
            for k in range(8):
                wrows[i, pl.ds(k * L, L)] = hrows[i, pl.ds(k * L, L)] * a
            return c2
        lax.fori_loop(0, C, wbody, 0)
        pltpu.sync_copy(wrows, u_sh.at[dst_loc.at[j]], add=True)
        return c_
    lax.fori_loop(0, kj, body_b, 0)

    plsc.subcore_barrier()

    pltpu.sync_copy(u_sh.at[pl.ds(sid * rows, rows)],
                    u_out.at[cid, pl.ds(sid * rows, rows)])
    pltpu.sync_copy(as_sh.at[pl.ds(sid * rows, rows)],
                    a_out.at[cid, pl.ds(sid * rows, rows)])


def kernel(node_states, edges, kernel, kernel_attention):
    n, d = node_states.shape
    u = kernel.shape[1]
    e = edges.shape[0]

    edges = edges.astype(jnp.int32)
    dst = edges[:, 0]
    src = edges[:, 1]

    rb = 512
    np_ = ((n + 1 + rb - 1) // rb) * rb
    kj = -(-e // (NW * C))
    ep = NW * kj * C

    ns_p = jnp.pad(node_states, ((0, np_ - n), (0, 0)))
    at = kernel_attention.reshape(2, u)
    dst_p = jnp.concatenate(
        [dst, jnp.full((ep - e,), n, jnp.int32)]).reshape(NW * kj, C)
    src_p = jnp.concatenate(
        [src, jnp.zeros((ep - e,), jnp.int32)]).reshape(NW * kj, C)

    h, st = pl.pallas_call(
        _tc_prep,
        grid=(np_ // rb,),
        in_specs=[
            pl.BlockSpec((rb, d), lambda i: (i, 0)),
            pl.BlockSpec((d, u), lambda i: (0, 0)),
            pl.BlockSpec((2, u), lambda i: (0, 0)),
        ],
        out_specs=[
            pl.BlockSpec((rb, u), lambda i: (i, 0)),
            pl.BlockSpec((2, rb), lambda i: (0, i)),
        ],
        out_shape=[
            jax.ShapeDtypeStruct((np_, u), jnp.float32),
            jax.ShapeDtypeStruct((2, np_), jnp.float32),
        ],
    )(ns_p, kernel, at)

    mesh = plsc.VectorSubcoreMesh(core_axis_name="c", subcore_axis_name="s")
    u_part, a_part = pl.kernel(
        functools.partial(_sc_body, kj=kj, np_=np_),
        out_type=[
            jax.ShapeDtypeStruct((NC, np_, u), jnp.float32),
            jax.ShapeDtypeStruct((NC, np_), jnp.float32),
        ],
        mesh=mesh,
        scratch_types=[
            pltpu.VMEM((np_,), jnp.float32),
            pltpu.VMEM((np_,), jnp.float32),
            pltpu.VMEM((kj, C), jnp.int32),
            pltpu.VMEM((kj, C), jnp.int32),
            pltpu.VMEM((kj, C), jnp.float32),
            pltpu.VMEM((C, u), jnp.float32),
            pltpu.VMEM((C, u), jnp.float32),
            pltpu.VMEM((L, u), jnp.float32),
            pltpu.VMEM((np_ // NS,), jnp.float32),
            pltpu.VMEM_SHARED((np_, u), jnp.float32),
            pltpu.VMEM_SHARED((np_,), jnp.float32),
            pltpu.SemaphoreType.DMA,
        ],
    )(h, st, dst_p, src_p)

    out = pl.pallas_call(
        _tc_finish,
        grid=(np_ // rb,),
        in_specs=[
            pl.BlockSpec((NC, rb, u), lambda i: (0, i, 0)),
            pl.BlockSpec((NC, rb), lambda i: (0, i)),
        ],
        out_specs=pl.BlockSpec((rb, u), lambda i: (i, 0)),
        out_shape=jax.ShapeDtypeStruct((np_, u), jnp.float32),
    )(u_part, a_part)

    return out[:n]


# speedup vs baseline: 8.6930x; 8.6930x over previous
import functools

import jax
import jax.numpy as jnp
from jax import lax
from jax.experimental import pallas as pl
from jax.experimental.pallas import tpu as pltpu
from jax.experimental.pallas import tpu_sc as plsc

L = 16
NC = 2
NS = 16
NW = NC * NS
C = 128


def _tc_prep(ns_ref, w_ref, at_ref, h_ref, s_ref, t_ref):
    h = jnp.dot(ns_ref[...], w_ref[...], preferred_element_type=jnp.float32)
    h_ref[...] = h
    st = lax.dot_general(
        at_ref[...], h, (((1,), (1,)), ((), ())),
        preferred_element_type=jnp.float32)
    s_ref[...] = st[0]
    t_ref[...] = st[1]


def _tc_finish(u_ref, as_ref, o_ref):
    u = u_ref[0] + u_ref[1]
    d = (as_ref[0] + as_ref[1])[:, None]
    o_ref[...] = jnp.where(d > 0.0, u / jnp.where(d > 0.0, d, 1.0), 0.0)


def _sc_body(h_hbm, s_hbm, t_hbm, dst_hbm, src_hbm, u_out, a_out,
             dst8, src8, att_b, sd_b, ts_b, hrows, wrows, u_sh, as_sh,
             *, kj, np_):
    cid = lax.axis_index("c")
    sid = lax.axis_index("s")
    w = cid * NS + sid
    rows = np_ // NS
    z16 = jnp.zeros((L,), jnp.float32)

    def zw(i, c_):
        for k in range(8):
            wrows[i, pl.ds(k * L, L)] = z16
        return c_
    lax.fori_loop(0, C, zw, 0)
    for k in range(C // L):
        sd_b[pl.ds(k * L, L)] = z16

    def zu(i, c_):
        pltpu.sync_copy(wrows, u_sh.at[pl.ds(sid * rows + i * C, C)])
        pltpu.sync_copy(sd_b, as_sh.at[pl.ds(sid * rows + i * C, C)])
        return c_
    lax.fori_loop(0, rows // C, zu, 0)

    plsc.subcore_barrier()

    def body(jb, c_):
        pltpu.sync_copy(dst_hbm.at[pl.ds(w * kj + jb * 8, 8)], dst8)
        pltpu.sync_copy(src_hbm.at[pl.ds(w * kj + jb * 8, 8)], src8)

        def chunk(j, c2):
            pltpu.sync_copy(s_hbm.at[dst8.at[j]], sd_b)
            pltpu.sync_copy(t_hbm.at[src8.at[j]], ts_b)

            def att_k(k, c3):
                x = sd_b[pl.ds(k * L, L)] + ts_b[pl.ds(k * L, L)]
                x = jnp.maximum(x, 0.2 * x)
                x = jnp.minimum(jnp.maximum(x, -2.0), 2.0)
                att_b[pl.ds(k * L, L)] = jnp.exp(x)
                return c3
            lax.fori_loop(0, C // L, att_k, 0)
            pltpu.sync_copy(att_b, as_sh.at[dst8.at[j]], add=True)

            pltpu.sync_copy(h_hbm.at[src8.at[j]], hrows)

            def wbody(j16, c3):
                attv = att_b[pl.ds(j16 * L, L)]
                for i16 in range(L):
                    a = attv[i16]
                    i = j16 * L + i16
                    for k in range(8):
                        wrows[i, pl.ds(k * L, L)] = (
                            hrows[i, pl.ds(k * L, L)] * a)
                return c3
            lax.fori_loop(0, C // L, wbody, 0)
            pltpu.sync_copy(wrows, u_sh.at[dst8.at[j]], add=True)
            return c2
        lax.fori_loop(0, 8, chunk, 0)
        return c_
    lax.fori_loop(0, kj // 8, body, 0)

    plsc.subcore_barrier()

    pltpu.sync_copy(u_sh.at[pl.ds(sid * rows, rows)],
                    u_out.at[cid, pl.ds(sid * rows, rows)])
    pltpu.sync_copy(as_sh.at[pl.ds(sid * rows, rows)],
                    a_out.at[cid, pl.ds(sid * rows, rows)])


def kernel(node_states, edges, kernel, kernel_attention):
    n, d = node_states.shape
    u = kernel.shape[1]
    e = edges.shape[0]

    edges = edges.astype(jnp.int32)
    dst = edges[:, 0]
    src = edges[:, 1]

    rb = 512
    np_ = ((n + 1 + rb - 1) // rb) * rb
    kj = ((-(-e // (NW * C)) + 7) // 8) * 8
    ep = NW * kj * C

    ns_p = jnp.pad(node_states, ((0, np_ - n), (0, 0)))
    at = kernel_attention.reshape(2, u)
    dst_p = jnp.concatenate(
        [dst, jnp.full((ep - e,), n, jnp.int32)]).reshape(NW * kj, C)
    src_p = jnp.concatenate(
        [src, jnp.zeros((ep - e,), jnp.int32)]).reshape(NW * kj, C)

    h, s, t = pl.pallas_call(
        _tc_prep,
        grid=(np_ // rb,),
        in_specs=[
            pl.BlockSpec((rb, d), lambda i: (i, 0)),
            pl.BlockSpec((d, u), lambda i: (0, 0)),
            pl.BlockSpec((2, u), lambda i: (0, 0)),
        ],
        out_specs=[
            pl.BlockSpec((rb, u), lambda i: (i, 0)),
            pl.BlockSpec((rb,), lambda i: (i,)),
            pl.BlockSpec((rb,), lambda i: (i,)),
        ],
        out_shape=[
            jax.ShapeDtypeStruct((np_, u), jnp.float32),
            jax.ShapeDtypeStruct((np_,), jnp.float32),
            jax.ShapeDtypeStruct((np_,), jnp.float32),
        ],
    )(ns_p, kernel, at)

    mesh = plsc.VectorSubcoreMesh(core_axis_name="c", subcore_axis_name="s")
    u_part, a_part = pl.kernel(
        functools.partial(_sc_body, kj=kj, np_=np_),
        out_type=[
            jax.ShapeDtypeStruct((NC, np_, u), jnp.float32),
            jax.ShapeDtypeStruct((NC, np_), jnp.float32),
        ],
        mesh=mesh,
        compiler_params=pltpu.CompilerParams(needs_layout_passes=False),
        scratch_types=[
            pltpu.VMEM((8, C), jnp.int32),
            pltpu.VMEM((8, C), jnp.int32),
            pltpu.VMEM((C,), jnp.float32),
            pltpu.VMEM((C,), jnp.float32),
            pltpu.VMEM((C,), jnp.float32),
            pltpu.VMEM((C, u), jnp.float32),
            pltpu.VMEM((C, u), jnp.float32),
            pltpu.VMEM_SHARED((np_, u), jnp.float32),
            pltpu.VMEM_SHARED((np_,), jnp.float32),
        ],
    )(h, s, t, dst_p, src_p)

    out = pl.pallas_call(
        _tc_finish,
        grid=(np_ // rb,),
        in_specs=[
            pl.BlockSpec((NC, rb, u), lambda i: (0, i, 0)),
            pl.BlockSpec((NC, rb), lambda i: (0, i)),
        ],
        out_specs=pl.BlockSpec((rb, u), lambda i: (i, 0)),
        out_shape=jax.ShapeDtypeStruct((np_, u), jnp.float32),
    )(u_part, a_part)

    return out[:n]


# speedup vs baseline: 13.4363x; 1.5456x over previous
import functools

import jax
import jax.numpy as jnp
from jax import lax
from jax.experimental import pallas as pl
from jax.experimental.pallas import tpu as pltpu
from jax.experimental.pallas import tpu_sc as plsc

L = 16
NC = 2
NS = 16
NW = NC * NS
C = 128


def _tc_prep(ns_ref, w_ref, at_ref, h_ref, s_ref, t_ref):
    h = jnp.dot(ns_ref[...], w_ref[...], preferred_element_type=jnp.float32)
    h_ref[...] = h
    st = lax.dot_general(
        at_ref[...], h, (((1,), (1,)), ((), ())),
        preferred_element_type=jnp.float32)
    s_ref[...] = st[0]
    t_ref[...] = st[1]


def _tc_finish(u_ref, as_ref, o_ref):
    u = u_ref[0] + u_ref[1]
    d = (as_ref[0] + as_ref[1])[:, None]
    o_ref[...] = jnp.where(d > 0.0, u / jnp.where(d > 0.0, d, 1.0), 0.0)


def _sc_body(h_hbm, s_hbm, t_hbm, dst_hbm, src_hbm, u_out, a_out,
             dst8, src8, att_b, sd_b, ts_b, hr,
             sem_h, sem_sd, sem_ts, sem_u, sem_a, sem_e, u_sh, as_sh,
             *, kj, np_):
    cid = lax.axis_index("c")
    sid = lax.axis_index("s")
    w = cid * NS + sid
    rows = np_ // NS
    z16 = jnp.zeros((L,), jnp.float32)

    def zw(i, c_):
        for k in range(8):
            hr[0, i, pl.ds(k * L, L)] = z16
        return c_
    lax.fori_loop(0, C, zw, 0)
    for k in range(C // L):
        sd_b[0, pl.ds(k * L, L)] = z16

    def zu(i, c_):
        pltpu.sync_copy(hr.at[0], u_sh.at[pl.ds(sid * rows + i * C, C)])
        pltpu.sync_copy(sd_b.at[0], as_sh.at[pl.ds(sid * rows + i * C, C)])
        return c_
    lax.fori_loop(0, rows // C, zu, 0)

    plsc.subcore_barrier()

    nblk = kj // 8

    def gathers(blk_buf, j, b):
        d_sd = pltpu.async_copy(
            s_hbm.at[dst8.at[blk_buf, j]], sd_b.at[b], sem_sd.at[b])
        d_ts = pltpu.async_copy(
            t_hbm.at[src8.at[blk_buf, j]], ts_b.at[b], sem_ts.at[b])
        d_h = pltpu.async_copy(
            h_hbm.at[src8.at[blk_buf, j]], hr.at[b], sem_h.at[b])
        return d_sd, d_ts, d_h

    pltpu.sync_copy(dst_hbm.at[pl.ds(w * kj, 8)], dst8.at[0])
    pltpu.sync_copy(src_hbm.at[pl.ds(w * kj, 8)], src8.at[0])

    def body(blk, c_):
        cur = lax.rem(blk, 2)
        nxt = lax.rem(blk + 1, 2)
        d_ed = d_es = None
        d_ed = pltpu.async_copy(
            dst_hbm.at[pl.ds(w * kj + lax.min(blk + 1, nblk - 1) * 8, 8)],
            dst8.at[nxt], sem_e.at[0])
        d_es = pltpu.async_copy(
            src_hbm.at[pl.ds(w * kj + lax.min(blk + 1, nblk - 1) * 8, 8)],
            src8.at[nxt], sem_e.at[1])

        pend = [None, None]
        d_g = [None, None]
        d_g[0] = gathers(cur, 0, 0)
        for j in range(8):
            b = j % 2
            if j < 7:
                if pend[1 - b] is not None:
                    pend[1 - b][0].wait()
                    pend[1 - b][1].wait()
                d_g[1 - b] = gathers(cur, j + 1, 1 - b)
            d_sd, d_ts, d_h = d_g[b]
            d_sd.wait()
            d_ts.wait()

            def att_k(k, c3, b=b):
                x = sd_b[b, pl.ds(k * L, L)] + ts_b[b, pl.ds(k * L, L)]
                x = jnp.maximum(x, 0.2 * x)
                x = jnp.minimum(jnp.maximum(x, -2.0), 2.0)
                att_b[b, pl.ds(k * L, L)] = jnp.exp(x)
                return c3
            lax.fori_loop(0, C // L, att_k, 0)
            d_sa = pltpu.async_copy(
                att_b.at[b], as_sh.at[dst8.at[cur, j]], sem_a.at[b],
                add=True)

            d_h.wait()

            def wbody(j16, c3, b=b):
                attv = att_b[b, pl.ds(j16 * L, L)]
                for i16 in range(L):
                    a = attv[i16]
                    i = j16 * L + i16
                    for k in range(8):
                        hr[b, i, pl.ds(k * L, L)] = (
                            hr[b, i, pl.ds(k * L, L)] * a)
                return c3
            lax.fori_loop(0, C // L, wbody, 0)
            d_su = pltpu.async_copy(
                hr.at[b], u_sh.at[dst8.at[cur, j]], sem_u.at[b], add=True)
            pend[b] = (d_su, d_sa)
        for b in range(2):
            pend[b][0].wait()
            pend[b][1].wait()
        d_ed.wait()
        d_es.wait()
        return c_
    lax.fori_loop(0, nblk, body, 0)

    plsc.subcore_barrier()

    pltpu.sync_copy(u_sh.at[pl.ds(sid * rows, rows)],
                    u_out.at[cid, pl.ds(sid * rows, rows)])
    pltpu.sync_copy(as_sh.at[pl.ds(sid * rows, rows)],
                    a_out.at[cid, pl.ds(sid * rows, rows)])


def kernel(node_states, edges, kernel, kernel_attention):
    n, d = node_states.shape
    u = kernel.shape[1]
    e = edges.shape[0]

    edges = edges.astype(jnp.int32)
    dst = edges[:, 0]
    src = edges[:, 1]

    rb = 512
    np_ = ((n + 1 + rb - 1) // rb) * rb
    kj = ((-(-e // (NW * C)) + 7) // 8) * 8
    ep = NW * kj * C

    ns_p = jnp.pad(node_states, ((0, np_ - n), (0, 0)))
    at = kernel_attention.reshape(2, u)
    dst_p = jnp.concatenate(
        [dst, jnp.full((ep - e,), n, jnp.int32)]).reshape(NW * kj, C)
    src_p = jnp.concatenate(
        [src, jnp.zeros((ep - e,), jnp.int32)]).reshape(NW * kj, C)

    h, s, t = pl.pallas_call(
        _tc_prep,
        grid=(np_ // rb,),
        in_specs=[
            pl.BlockSpec((rb, d), lambda i: (i, 0)),
            pl.BlockSpec((d, u), lambda i: (0, 0)),
            pl.BlockSpec((2, u), lambda i: (0, 0)),
        ],
        out_specs=[
            pl.BlockSpec((rb, u), lambda i: (i, 0)),
            pl.BlockSpec((rb,), lambda i: (i,)),
            pl.BlockSpec((rb,), lambda i: (i,)),
        ],
        out_shape=[
            jax.ShapeDtypeStruct((np_, u), jnp.float32),
            jax.ShapeDtypeStruct((np_,), jnp.float32),
            jax.ShapeDtypeStruct((np_,), jnp.float32),
        ],
    )(ns_p, kernel, at)

    mesh = plsc.VectorSubcoreMesh(core_axis_name="c", subcore_axis_name="s")
    u_part, a_part = pl.kernel(
        functools.partial(_sc_body, kj=kj, np_=np_),
        out_type=[
            jax.ShapeDtypeStruct((NC, np_, u), jnp.float32),
            jax.ShapeDtypeStruct((NC, np_), jnp.float32),
        ],
        mesh=mesh,
        compiler_params=pltpu.CompilerParams(needs_layout_passes=False),
        scratch_types=[
            pltpu.VMEM((2, 8, C), jnp.int32),
            pltpu.VMEM((2, 8, C), jnp.int32),
            pltpu.VMEM((2, C), jnp.float32),
            pltpu.VMEM((2, C), jnp.float32),
            pltpu.VMEM((2, C), jnp.float32),
            pltpu.VMEM((2, C, u), jnp.float32),
            pltpu.SemaphoreType.DMA((2,)),
            pltpu.SemaphoreType.DMA((2,)),
            pltpu.SemaphoreType.DMA((2,)),
            pltpu.SemaphoreType.DMA((2,)),
            pltpu.SemaphoreType.DMA((2,)),
            pltpu.SemaphoreType.DMA((2,)),
            pltpu.VMEM_SHARED((np_, u), jnp.float32),
            pltpu.VMEM_SHARED((np_,), jnp.float32),
        ],
    )(h, s, t, dst_p, src_p)

    out = pl.pallas_call(
        _tc_finish,
        grid=(np_ // rb,),
        in_specs=[
            pl.BlockSpec((NC, rb, u), lambda i: (0, i, 0)),
            pl.BlockSpec((NC, rb), lambda i: (0, i)),
        ],
        out_specs=pl.BlockSpec((rb, u), lambda i: (i, 0)),
        out_shape=jax.ShapeDtypeStruct((np_, u), jnp.float32),
    )(u_part, a_part)

    return out[:n]
